# baseline (speedup 1.0000x reference)
import functools

import jax
import jax.numpy as jnp
from jax import lax
from jax.experimental import pallas as pl
from jax.experimental.pallas import tpu as pltpu
from jax.experimental.pallas import tpu_sc as plsc

_T, _B, _L, _V, _D = 26, 4096, 20, 1000, 128
_NB = _T * _B
_NC, _NS = 2, 16
_NW = _NC * _NS
_BW = _NB // _NW
_CB = 16
_NQ = _D // 16
_NP = _D // 32
_WR = _D // 2
_RS = 200
_ILV = plsc.PackFormat.INTERLEAVED


def _accum_chunk(idx_v, table_i, pooled_v):
    def bag(j, _):
        base = j * _L
        w0 = idx_v[pl.ds(base, 16)]
        w1 = idx_v[pl.ds(base + 16, 16)]
        rs = [w0[l] for l in range(16)] + [w1[l] for l in range(_L - 16)]
        for p in range(_NP):
            vals = [plsc.bitcast(table_i[pl.ds(rs[l] * _WR + 16 * p, 16)],
                                 jnp.bfloat16) for l in range(_L)]
            while len(vals) > 1:
                nxt = [vals[i] + vals[i + 1] for i in range(0, len(vals) - 1, 2)]
                if len(vals) % 2:
                    nxt.append(vals[-1])
                vals = nxt
            a, b = plsc.unpack(vals[0], format=_ILV)
            pooled_v[j, pl.ds(p * 32, 16)] = a
            pooled_v[j, pl.ds(p * 32 + 16, 16)] = b
        return 0

    lax.fori_loop(0, _CB, bag, 0, unroll=False)


def _stage_table(t, w_hbm, stage_v, table_i):
    for c in range(_V // _RS):
        off = pl.multiple_of(t * _V + c * _RS, 8)
        pltpu.sync_copy(w_hbm.at[pl.ds(off, _RS)], stage_v)

        def row(i, _):
            for p in range(_NP):
                a = stage_v[i, pl.ds(32 * p, 16)]
                b = stage_v[i, pl.ds(32 * p + 16, 16)]
                table_i[pl.ds((c * _RS + i) * _WR + 16 * p, 16)] = (
                    plsc.bitcast(plsc.pack(a, b, format=_ILV), jnp.int32))
            return 0

        lax.fori_loop(0, _RS, row, 0, unroll=False)


def _emb_body(idx_hbm, w_hbm, out_hbm, table_i, stage_v, idx0, idx1,
              pool0, pool1, isem0, isem1, osem0, osem1):
    cid = lax.axis_index("c")
    sid = lax.axis_index("s")
    wid = cid * _NS + sid
    s = wid * _BW

    def idx_dma(bag0, buf, sem):
        off = pl.multiple_of(bag0 * _L, 8 * _L)
        return pltpu.async_copy(idx_hbm.at[pl.ds(off, _CB * _L)],
                                buf.at[pl.ds(0, _CB * _L)], sem)

    def idx_wait(buf, sem):
        pltpu.make_async_copy(idx_hbm.at[pl.ds(0, _CB * _L)],
                              buf.at[pl.ds(0, _CB * _L)], sem).wait()

    def out_dma(pool, bag0, sem):
        off = pl.multiple_of(bag0, 8)
        return pltpu.async_copy(pool, out_hbm.at[pl.ds(off, _CB)], sem)

    def out_wait(pool, sem):
        pltpu.make_async_copy(pool, out_hbm.at[pl.ds(0, _CB)], sem).wait()

    def phase(t, _):
        b_lo = jnp.maximum(s, t * _B)
        b_hi = jnp.minimum(s + _BW, (t + 1) * _B)
        npair = (b_hi - b_lo) // (2 * _CB)

        @pl.when(npair > 0)
        def _():
            _stage_table(t, w_hbm, stage_v, table_i)
            idx_dma(b_lo, idx0, isem0)

            def pair(k, _):
                bag_a = b_lo + (2 * k) * _CB
                bag_b = bag_a + _CB
                idx_wait(idx0, isem0)
                idx_dma(bag_b, idx1, isem1)

                @pl.when(k >= 1)
                def _():
                    out_wait(pool0, osem0)

                _accum_chunk(idx0, table_i, pool0)
                out_dma(pool0, bag_a, osem0)
                idx_wait(idx1, isem1)

                @pl.when(k + 1 < npair)
                def _():
                    idx_dma(bag_b + _CB, idx0, isem0)

                @pl.when(k >= 1)
                def _():
                    out_wait(pool1, osem1)

                _accum_chunk(idx1, table_i, pool1)
                out_dma(pool1, bag_b, osem1)
                return 0

            lax.fori_loop(0, npair, pair, 0, unroll=False)
            out_wait(pool0, osem0)
            out_wait(pool1, osem1)

        return 0

    t0 = s // _B
    lax.fori_loop(t0, t0 + 2, phase, 0, unroll=False)


@functools.partial(jax.jit, static_argnames=())
def kernel(indices, offsets, weight):
    del offsets
    mesh = plsc.VectorSubcoreMesh(
        core_axis_name="c", subcore_axis_name="s",
        num_cores=_NC, num_subcores=_NS)
    run = pl.kernel(
        _emb_body,
        out_type=jax.ShapeDtypeStruct((_NB, _D), jnp.float32),
        mesh=mesh,
        compiler_params=pltpu.CompilerParams(needs_layout_passes=False),
        scratch_types=[
            pltpu.VMEM((_V * _WR,), jnp.int32),
            pltpu.VMEM((_RS, _D), jnp.float32),
            pltpu.VMEM((_CB * _L + 16,), jnp.int32),
            pltpu.VMEM((_CB * _L + 16,), jnp.int32),
            pltpu.VMEM((_CB, _D), jnp.float32),
            pltpu.VMEM((_CB, _D), jnp.float32),
            pltpu.SemaphoreType.DMA,
            pltpu.SemaphoreType.DMA,
            pltpu.SemaphoreType.DMA,
            pltpu.SemaphoreType.DMA,
        ],
    )
    pooled = run(indices, weight)
    return pooled.reshape(_T, _B, _D)


# speedup vs baseline: 1.0595x; 1.0595x over previous
import functools

import jax
import jax.numpy as jnp
from jax import lax
from jax.experimental import pallas as pl
from jax.experimental.pallas import tpu as pltpu
from jax.experimental.pallas import tpu_sc as plsc

_T, _B, _L, _V, _D = 26, 4096, 20, 1000, 128
_NB = _T * _B
_NC, _NS = 2, 16
_NW = _NC * _NS
_BW = _NB // _NW
_CB = 16
_NQ = _D // 16
_NP = _D // 32
_WR = _D // 2
_RS = 200
_ILV = plsc.PackFormat.INTERLEAVED


def _accum_chunk(idx_v, table_i, pooled_v):
    def bagpair(m, _):
        rss = []
        for j2 in range(2):
            base = (2 * m + j2) * _L
            w0 = idx_v[pl.ds(base, 16)]
            w1 = idx_v[pl.ds(base + 16, 16)]
            rss.append([w0[l] for l in range(16)]
                       + [w1[l] for l in range(_L - 16)])
        for j2 in range(2):
            j, rs = 2 * m + j2, rss[j2]
            for p in range(_NP):
                vals = [plsc.bitcast(table_i[pl.ds(rs[l] * _WR + 16 * p, 16)],
                                     jnp.bfloat16) for l in range(_L)]
                while len(vals) > 1:
                    nxt = [vals[i] + vals[i + 1]
                           for i in range(0, len(vals) - 1, 2)]
                    if len(vals) % 2:
                        nxt.append(vals[-1])
                    vals = nxt
                a, b = plsc.unpack(vals[0], format=_ILV)
                pooled_v[j, pl.ds(p * 32, 16)] = a
                pooled_v[j, pl.ds(p * 32 + 16, 16)] = b
        return 0

    lax.fori_loop(0, _CB // 2, bagpair, 0, unroll=False)


def _stage_table(t, w_hbm, stage_v, table_i):
    for c in range(_V // _RS):
        off = pl.multiple_of(t * _V + c * _RS, 8)
        pltpu.sync_copy(w_hbm.at[pl.ds(off, _RS)], stage_v)

        def row(i, _):
            for p in range(_NP):
                a = stage_v[i, pl.ds(32 * p, 16)]
                b = stage_v[i, pl.ds(32 * p + 16, 16)]
                table_i[pl.ds((c * _RS + i) * _WR + 16 * p, 16)] = (
                    plsc.bitcast(plsc.pack(a, b, format=_ILV), jnp.int32))
            return 0

        lax.fori_loop(0, _RS, row, 0, unroll=False)


def _emb_body(idx_hbm, w_hbm, out_hbm, table_i, stage_v, idx0, idx1,
              pool0, pool1, isem0, isem1, osem0, osem1):
    cid = lax.axis_index("c")
    sid = lax.axis_index("s")
    wid = cid * _NS + sid
    s = wid * _BW

    def idx_dma(bag0, buf, sem):
        off = pl.multiple_of(bag0 * _L, 8 * _L)
        return pltpu.async_copy(idx_hbm.at[pl.ds(off, _CB * _L)],
                                buf.at[pl.ds(0, _CB * _L)], sem)

    def idx_wait(buf, sem):
        pltpu.make_async_copy(idx_hbm.at[pl.ds(0, _CB * _L)],
                              buf.at[pl.ds(0, _CB * _L)], sem).wait()

    def out_dma(pool, bag0, sem):
        off = pl.multiple_of(bag0, 8)
        return pltpu.async_copy(pool, out_hbm.at[pl.ds(off, _CB)], sem)

    def out_wait(pool, sem):
        pltpu.make_async_copy(pool, out_hbm.at[pl.ds(0, _CB)], sem).wait()

    def phase(t, _):
        b_lo = jnp.maximum(s, t * _B)
        b_hi = jnp.minimum(s + _BW, (t + 1) * _B)
        npair = (b_hi - b_lo) // (2 * _CB)

        @pl.when(npair > 0)
        def _():
            _stage_table(t, w_hbm, stage_v, table_i)
            idx_dma(b_lo, idx0, isem0)

            def pair(k, _):
                bag_a = b_lo + (2 * k) * _CB
                bag_b = bag_a + _CB
                idx_wait(idx0, isem0)
                idx_dma(bag_b, idx1, isem1)

                @pl.when(k >= 1)
                def _():
                    out_wait(pool0, osem0)

                _accum_chunk(idx0, table_i, pool0)
                out_dma(pool0, bag_a, osem0)
                idx_wait(idx1, isem1)

                @pl.when(k + 1 < npair)
                def _():
                    idx_dma(bag_b + _CB, idx0, isem0)

                @pl.when(k >= 1)
                def _():
                    out_wait(pool1, osem1)

                _accum_chunk(idx1, table_i, pool1)
                out_dma(pool1, bag_b, osem1)
                return 0

            lax.fori_loop(0, npair, pair, 0, unroll=False)
            out_wait(pool0, osem0)
            out_wait(pool1, osem1)

        return 0

    t0 = s // _B
    lax.fori_loop(t0, t0 + 2, phase, 0, unroll=False)


@functools.partial(jax.jit, static_argnames=())
def kernel(indices, offsets, weight):
    del offsets
    mesh = plsc.VectorSubcoreMesh(
        core_axis_name="c", subcore_axis_name="s",
        num_cores=_NC, num_subcores=_NS)
    run = pl.kernel(
        _emb_body,
        out_type=jax.ShapeDtypeStruct((_NB, _D), jnp.float32),
        mesh=mesh,
        compiler_params=pltpu.CompilerParams(needs_layout_passes=False),
        scratch_types=[
            pltpu.VMEM((_V * _WR,), jnp.int32),
            pltpu.VMEM((_RS, _D), jnp.float32),
            pltpu.VMEM((_CB * _L + 16,), jnp.int32),
            pltpu.VMEM((_CB * _L + 16,), jnp.int32),
            pltpu.VMEM((_CB, _D), jnp.float32),
            pltpu.VMEM((_CB, _D), jnp.float32),
            pltpu.SemaphoreType.DMA,
            pltpu.SemaphoreType.DMA,
            pltpu.SemaphoreType.DMA,
            pltpu.SemaphoreType.DMA,
        ],
    )
    pooled = run(indices, weight)
    return pooled.reshape(_T, _B, _D)


# speedup vs baseline: 2.0329x; 1.9187x over previous
import functools

import jax
import jax.numpy as jnp
from jax import lax
from jax.experimental import pallas as pl
from jax.experimental.pallas import tpu as pltpu
from jax.experimental.pallas import tpu_sc as plsc

_T, _B, _L, _V, _D = 26, 4096, 20, 1000, 128
_NB = _T * _B
_NC, _NS = 2, 16
_NW = _NC * _NS
_BW = _NB // _NW
_CB = 16
_NQ = _D // 16
_NP = _D // 32
_WR = _D // 2
_RS = 200
_ILV = plsc.PackFormat.INTERLEAVED


def _accum_chunk(idx_v, table_i, pooled_v):
    @functools.partial(plsc.parallel_loop, 0, _CB, unroll=2)
    def bag(j):
        base = j * _L
        w0 = idx_v[pl.ds(base, 16)]
        w1 = idx_v[pl.ds(base + 16, 16)]
        rs = [w0[l] for l in range(16)] + [w1[l] for l in range(_L - 16)]
        for p in range(_NP):
            vals = [plsc.bitcast(table_i[pl.ds(rs[l] * _WR + 16 * p, 16)],
                                 jnp.bfloat16) for l in range(_L)]
            while len(vals) > 1:
                nxt = [vals[i] + vals[i + 1]
                       for i in range(0, len(vals) - 1, 2)]
                if len(vals) % 2:
                    nxt.append(vals[-1])
                vals = nxt
            a, b = plsc.unpack(vals[0], format=_ILV)
            pooled_v[j, pl.ds(p * 32, 16)] = a
            pooled_v[j, pl.ds(p * 32 + 16, 16)] = b


def _stage_table(t, w_hbm, stage_v, table_i):
    for c in range(_V // _RS):
        off = pl.multiple_of(t * _V + c * _RS, 8)
        pltpu.sync_copy(w_hbm.at[pl.ds(off, _RS)], stage_v)

        def row(i, _):
            for p in range(_NP):
                a = stage_v[i, pl.ds(32 * p, 16)]
                b = stage_v[i, pl.ds(32 * p + 16, 16)]
                table_i[pl.ds((c * _RS + i) * _WR + 16 * p, 16)] = (
                    plsc.bitcast(plsc.pack(a, b, format=_ILV), jnp.int32))
            return 0

        lax.fori_loop(0, _RS, row, 0, unroll=False)


def _emb_body(idx_hbm, w_hbm, out_hbm, table_i, stage_v, idx0, idx1,
              pool0, pool1, isem0, isem1, osem0, osem1):
    cid = lax.axis_index("c")
    sid = lax.axis_index("s")
    wid = cid * _NS + sid
    s = wid * _BW

    def idx_dma(bag0, buf, sem):
        off = pl.multiple_of(bag0 * _L, 8 * _L)
        return pltpu.async_copy(idx_hbm.at[pl.ds(off, _CB * _L)],
                                buf.at[pl.ds(0, _CB * _L)], sem)

    def idx_wait(buf, sem):
        pltpu.make_async_copy(idx_hbm.at[pl.ds(0, _CB * _L)],
                              buf.at[pl.ds(0, _CB * _L)], sem).wait()

    def out_dma(pool, bag0, sem):
        off = pl.multiple_of(bag0, 8)
        return pltpu.async_copy(pool, out_hbm.at[pl.ds(off, _CB)], sem)

    def out_wait(pool, sem):
        pltpu.make_async_copy(pool, out_hbm.at[pl.ds(0, _CB)], sem).wait()

    def phase(t, _):
        b_lo = jnp.maximum(s, t * _B)
        b_hi = jnp.minimum(s + _BW, (t + 1) * _B)
        npair = (b_hi - b_lo) // (2 * _CB)

        @pl.when(npair > 0)
        def _():
            _stage_table(t, w_hbm, stage_v, table_i)
            idx_dma(b_lo, idx0, isem0)

            def pair(k, _):
                bag_a = b_lo + (2 * k) * _CB
                bag_b = bag_a + _CB
                idx_wait(idx0, isem0)
                idx_dma(bag_b, idx1, isem1)

                @pl.when(k >= 1)
                def _():
                    out_wait(pool0, osem0)

                _accum_chunk(idx0, table_i, pool0)
                out_dma(pool0, bag_a, osem0)
                idx_wait(idx1, isem1)

                @pl.when(k + 1 < npair)
                def _():
                    idx_dma(bag_b + _CB, idx0, isem0)

                @pl.when(k >= 1)
                def _():
                    out_wait(pool1, osem1)

                _accum_chunk(idx1, table_i, pool1)
                out_dma(pool1, bag_b, osem1)
                return 0

            lax.fori_loop(0, npair, pair, 0, unroll=False)
            out_wait(pool0, osem0)
            out_wait(pool1, osem1)

        return 0

    t0 = s // _B
    lax.fori_loop(t0, t0 + 2, phase, 0, unroll=False)


@functools.partial(jax.jit, static_argnames=())
def kernel(indices, offsets, weight):
    del offsets
    mesh = plsc.VectorSubcoreMesh(
        core_axis_name="c", subcore_axis_name="s",
        num_cores=_NC, num_subcores=_NS)
    run = pl.kernel(
        _emb_body,
        out_type=jax.ShapeDtypeStruct((_NB, _D), jnp.float32),
        mesh=mesh,
        compiler_params=pltpu.CompilerParams(needs_layout_passes=False),
        scratch_types=[
            pltpu.VMEM((_V * _WR,), jnp.int32),
            pltpu.VMEM((_RS, _D), jnp.float32),
            pltpu.VMEM((_CB * _L + 16,), jnp.int32),
            pltpu.VMEM((_CB * _L + 16,), jnp.int32),
            pltpu.VMEM((_CB, _D), jnp.float32),
            pltpu.VMEM((_CB, _D), jnp.float32),
            pltpu.SemaphoreType.DMA,
            pltpu.SemaphoreType.DMA,
            pltpu.SemaphoreType.DMA,
            pltpu.SemaphoreType.DMA,
        ],
    )
    pooled = run(indices, weight)
    return pooled.reshape(_T, _B, _D)
